# baseline (speedup 1.0000x reference)
import functools

import jax
import jax.numpy as jnp
from jax import lax
from jax.experimental import pallas as pl
from jax.experimental.pallas import tpu as pltpu
from jax.experimental.pallas import tpu_sc as plsc

N = 10000
E = 320000
D_IN = 128
D_HID = 16
D_PAD = 16

NUM_CORES = 2
NUM_SUBCORES = 16
NW = NUM_CORES * NUM_SUBCORES

CHUNK = 128
NBUF = 4
K = NBUF * (-(-E // (NW * CHUNK * NBUF)))
G = K // NBUF
EW = NW * K * CHUNK
assert G % 2 == 0
N_PAD = ((N + NW * 8 - 1) // (NW * 8)) * (NW * 8) + NW * 8
RPT = N_PAD // NUM_SUBCORES

_mesh = plsc.VectorSubcoreMesh(core_axis_name="c", subcore_axis_name="s")


@functools.partial(
    pl.kernel,
    mesh=_mesh,
    out_type=[
        jax.ShapeDtypeStruct((NW * RPT,), jnp.float32),
        jax.ShapeDtypeStruct((NW * RPT, D_PAD), jnp.float32),
    ],
    scratch_types=[
        pltpu.VMEM((K, CHUNK), jnp.int32),
        pltpu.VMEM((2 * K, CHUNK), jnp.int32),
        pltpu.VMEM((CHUNK,), jnp.float32),
        pltpu.VMEM((RPT,), jnp.float32),
        pltpu.VMEM((2 * NBUF, CHUNK, D_PAD), jnp.float32),
        pltpu.VMEM((RPT, D_PAD), jnp.float32),
        pltpu.VMEM_SHARED((N_PAD,), jnp.float32),
        pltpu.VMEM_SHARED((N_PAD, D_PAD), jnp.float32),
        pltpu.VMEM_SHARED((N_PAD, D_PAD), jnp.float32),
        pltpu.SemaphoreType.DMA((2 * NBUF,)),
        pltpu.SemaphoreType.DMA,
        pltpu.SemaphoreType.DMA,
    ],
    compiler_params=pltpu.CompilerParams(use_tc_tiling_on_sc=False),
)
def _deg_agg_kernel(src_hbm, dst_hbm, h1_hbm, deg_hbm, out_hbm,
                    src_v, dst_v, ones_v, dval_v, rows_v, buf_v,
                    deg_sh, tab_sh, acc_sh, sem, dsem, tsem):
    c = lax.axis_index("c")
    s = lax.axis_index("s")
    w = c * NUM_SUBCORES + s
    soff = pl.multiple_of(s * RPT, 8)
    woff = pl.multiple_of(w * RPT, 8)

    pltpu.async_copy(h1_hbm.at[pl.ds(soff, RPT), :],
                     tab_sh.at[pl.ds(soff, RPT), :], tsem)

    pltpu.sync_copy(src_hbm.at[w], src_v)
    pltpu.sync_copy(dst_hbm.at[s], dst_v.at[pl.ds(0, K)])
    pltpu.sync_copy(dst_hbm.at[s + NUM_SUBCORES], dst_v.at[pl.ds(K, K)])
    for i in range(CHUNK // 16):
        ones_v[pl.ds(i * 16, 16)] = jnp.ones((16,), jnp.float32)

    def dzfill(i, _):
        dval_v[pl.ds(i * 16, 16)] = jnp.zeros((16,), jnp.float32)
        return _
    lax.fori_loop(0, RPT // 16, dzfill, None)
    pltpu.sync_copy(dval_v, deg_sh.at[pl.ds(soff, RPT)])

    def zfill(i, _):
        buf_v[i, :] = jnp.zeros((16,), jnp.float32)
        return _
    lax.fori_loop(0, RPT, zfill, None)
    pltpu.sync_copy(buf_v, acc_sh.at[pl.ds(soff, RPT), :])
    plsc.subcore_barrier()

    def dchunk(j, _):
        pltpu.async_copy(ones_v, deg_sh.at[dst_v.at[j]], dsem, add=True)

        @pl.when(j >= 8)
        def _():
            pltpu.make_async_copy(ones_v, deg_sh.at[dst_v.at[j - 8]], dsem).wait()
        return _
    lax.fori_loop(0, 2 * K, dchunk, None)
    for t in range(8):
        pltpu.make_async_copy(ones_v, deg_sh.at[dst_v.at[2 * K - 8 + t]],
                              dsem).wait()
    plsc.subcore_barrier()

    pltpu.sync_copy(deg_sh.at[pl.ds(soff, RPT)], dval_v)
    pltpu.sync_copy(dval_v, deg_hbm.at[pl.ds(woff, RPT)])

    def dis_fill(i, _):
        x = dval_v[pl.ds(i * 16, 16)] + 1.0
        ix = lax.bitcast_convert_type(x, jnp.int32)
        iy = jnp.int32(0x5F3759DF) - lax.shift_right_arithmetic(ix, 1)
        y = lax.bitcast_convert_type(iy, jnp.float32)
        for _t in range(4):
            y = y * (1.5 - 0.5 * x * y * y)
        dval_v[pl.ds(i * 16, 16)] = y
        return _
    lax.fori_loop(0, RPT // 16, dis_fill, None)

    pltpu.make_async_copy(h1_hbm.at[pl.ds(soff, RPT), :],
                          tab_sh.at[pl.ds(soff, RPT), :], tsem).wait()
    pltpu.sync_copy(tab_sh.at[pl.ds(soff, RPT), :], buf_v)

    def scale_block(i, _):
        dvec = dval_v[pl.ds(i * 16, 16)]
        for t in range(16):
            r = i * 16 + t
            buf_v[r, :] = buf_v[r, :] * dvec[t]
        return _
    lax.fori_loop(0, RPT // 16, scale_block, None)
    pltpu.sync_copy(buf_v, tab_sh.at[pl.ds(soff, RPT), :])
    plsc.subcore_barrier()

    def fire_g(j, b):
        pltpu.async_copy(tab_sh.at[src_v.at[j]], rows_v.at[b], sem.at[b])

    def wait_g(j, b):
        pltpu.make_async_copy(tab_sh.at[src_v.at[j]], rows_v.at[b],
                              sem.at[b]).wait()

    def fire_s(j, b):
        pltpu.async_copy(rows_v.at[b], acc_sh.at[dst_v.at[c * K + j]],
                         sem.at[b], add=True)

    def wait_s(j, b):
        pltpu.make_async_copy(rows_v.at[b], acc_sh.at[dst_v.at[c * K + j]],
                              sem.at[b]).wait()

    for b in range(NBUF):
        fire_g(b, b)

    def group2(i2, _):
        for p in (0, 1):
            gi = 2 * i2 + p
            pb = NBUF * p
            ob = NBUF * (1 - p)
            for b in range(NBUF):
                wait_g(gi * NBUF + b, pb + b)
            for b in range(NBUF):
                fire_s(gi * NBUF + b, pb + b)
            for b in range(NBUF):
                @pl.when(gi >= 1)
                def _(jp=(gi - 1) * NBUF + b, bb=ob + b):
                    wait_s(jp, bb)
            for b in range(NBUF):
                @pl.when(gi + 1 < G)
                def _(jn=(gi + 1) * NBUF + b, bb=ob + b):
                    fire_g(jn, bb)
        return _
    lax.fori_loop(0, G // 2, group2, None)
    for b in range(NBUF):
        wait_s((G - 1) * NBUF + b, NBUF * ((G - 1) % 2) + b)
    plsc.subcore_barrier()

    pltpu.sync_copy(acc_sh.at[pl.ds(soff, RPT), :], buf_v)
    pltpu.sync_copy(buf_v, out_hbm.at[pl.ds(woff, RPT), :])


@functools.partial(
    pl.kernel,
    mesh=_mesh,
    out_type=jax.ShapeDtypeStruct((NW * RPT, D_PAD), jnp.float32),
    scratch_types=[
        pltpu.VMEM((K, CHUNK), jnp.int32),
        pltpu.VMEM((K, CHUNK), jnp.int32),
        pltpu.VMEM((2 * NBUF, CHUNK, D_PAD), jnp.float32),
        pltpu.VMEM((RPT, D_PAD), jnp.float32),
        pltpu.VMEM_SHARED((N_PAD, D_PAD), jnp.float32),
        pltpu.VMEM_SHARED((N_PAD, D_PAD), jnp.float32),
        pltpu.SemaphoreType.DMA((2 * NBUF,)),
        pltpu.SemaphoreType.DMA,
    ],
    compiler_params=pltpu.CompilerParams(use_tc_tiling_on_sc=False),
)
def _agg_kernel(src_hbm, dst_hbm, table_hbm, out_hbm,
                src_v, dst_v, rows_v, buf_v, tab_sh, acc_sh, sem, tsem):
    c = lax.axis_index("c")
    s = lax.axis_index("s")
    w = c * NUM_SUBCORES + s
    soff = pl.multiple_of(s * RPT, 8)
    woff = pl.multiple_of(w * RPT, 8)

    pltpu.async_copy(table_hbm.at[pl.ds(soff, RPT), :],
                     tab_sh.at[pl.ds(soff, RPT), :], tsem)

    pltpu.sync_copy(src_hbm.at[w], src_v)
    pltpu.sync_copy(dst_hbm.at[w], dst_v)

    def zfill(i, _):
        buf_v[i, :] = jnp.zeros((16,), jnp.float32)
        return _
    lax.fori_loop(0, RPT, zfill, None)
    pltpu.sync_copy(buf_v, acc_sh.at[pl.ds(soff, RPT), :])
    pltpu.make_async_copy(table_hbm.at[pl.ds(soff, RPT), :],
                          tab_sh.at[pl.ds(soff, RPT), :], tsem).wait()
    plsc.subcore_barrier()

    def fire_g(j, b):
        pltpu.async_copy(tab_sh.at[src_v.at[j]], rows_v.at[b], sem.at[b])

    def wait_g(j, b):
        pltpu.make_async_copy(tab_sh.at[src_v.at[j]], rows_v.at[b],
                              sem.at[b]).wait()

    def fire_s(j, b):
        pltpu.async_copy(rows_v.at[b], acc_sh.at[dst_v.at[j]], sem.at[b],
                         add=True)

    def wait_s(j, b):
        pltpu.make_async_copy(rows_v.at[b], acc_sh.at[dst_v.at[j]],
                              sem.at[b]).wait()

    for b in range(NBUF):
        fire_g(b, b)

    def group2(i2, _):
        for p in (0, 1):
            gi = 2 * i2 + p
            pb = NBUF * p
            ob = NBUF * (1 - p)
            for b in range(NBUF):
                wait_g(gi * NBUF + b, pb + b)
            for b in range(NBUF):
                fire_s(gi * NBUF + b, pb + b)
            for b in range(NBUF):
                @pl.when(gi >= 1)
                def _(jp=(gi - 1) * NBUF + b, bb=ob + b):
                    wait_s(jp, bb)
            for b in range(NBUF):
                @pl.when(gi + 1 < G)
                def _(jn=(gi + 1) * NBUF + b, bb=ob + b):
                    fire_g(jn, bb)
        return _
    lax.fori_loop(0, G // 2, group2, None)
    for b in range(NBUF):
        wait_s((G - 1) * NBUF + b, NBUF * ((G - 1) % 2) + b)
    plsc.subcore_barrier()

    pltpu.sync_copy(acc_sh.at[pl.ds(soff, RPT), :], buf_v)
    pltpu.sync_copy(buf_v, out_hbm.at[pl.ds(woff, RPT), :])


def _tc1_body(x_ref, w_ref, h_ref):
    h_ref[...] = jnp.dot(x_ref[...], w_ref[...],
                         preferred_element_type=jnp.float32)


def _tc2_body(acc_ref, h1_ref, deg_ref, w2_ref, b1_ref, g2_ref):
    deg = deg_ref[0] + 1.0
    dis = lax.rsqrt(deg)
    g1 = h1_ref[...] * dis
    out1 = dis * (acc_ref[0] + acc_ref[1] + g1) + b1_ref[...]
    h = jnp.maximum(out1, 0.0)
    g2_ref[...] = jnp.dot(h, w2_ref[...], preferred_element_type=jnp.float32) * dis


def _tc3_body(acc_ref, g2_ref, deg_ref, b2_ref, o_ref):
    deg = deg_ref[0] + 1.0
    dis = lax.rsqrt(deg)
    o = dis * (acc_ref[0] + acc_ref[1] + g2_ref[...]) + b2_ref[...]
    col = lax.broadcasted_iota(jnp.int32, o.shape, 1)
    mask = col < 5
    om = jnp.where(mask, o, -1e30)
    m = jnp.max(om, axis=1, keepdims=True)
    e = jnp.where(mask, jnp.exp(o - m), 0.0)
    ssum = jnp.sum(e, axis=1, keepdims=True)
    o_ref[...] = o - m - jnp.log(ssum)


_f32 = jnp.float32


@jax.jit
def kernel(x, edge_index, W1, b1, W2, b2):
    pad_e = EW - E
    src = jnp.concatenate([edge_index[0], jnp.full((pad_e,), N, jnp.int32)])
    dst = jnp.concatenate([edge_index[1], jnp.full((pad_e,), N, jnp.int32)])
    src3 = src.reshape(NW, K, CHUNK)
    dst3 = dst.reshape(NW, K, CHUNK)

    xp = jnp.zeros((N_PAD, D_IN), _f32).at[:N].set(x)
    W2p = jnp.zeros((D_HID, D_PAD), _f32).at[:, :W2.shape[1]].set(W2)
    b2p = jnp.zeros((D_PAD,), _f32).at[:W2.shape[1]].set(b2)

    h1p = pl.pallas_call(
        _tc1_body,
        out_shape=jax.ShapeDtypeStruct((N_PAD, D_HID), _f32),
    )(xp, W1)

    degp, acc1f = _deg_agg_kernel(src3, dst3, h1p)
    degp3 = degp.reshape(NUM_CORES, N_PAD, 1)
    acc1p = acc1f.reshape(NUM_CORES, N_PAD, D_PAD)

    g2p = pl.pallas_call(
        _tc2_body,
        out_shape=jax.ShapeDtypeStruct((N_PAD, D_PAD), _f32),
    )(acc1p, h1p, degp3, W2p, b1)

    acc2p = _agg_kernel(src3, dst3, g2p).reshape(NUM_CORES, N_PAD, D_PAD)

    outp = pl.pallas_call(
        _tc3_body,
        out_shape=jax.ShapeDtypeStruct((N_PAD, D_PAD), _f32),
    )(acc2p, g2p, degp3, b2p)

    return outp[:N, :W2.shape[1]]


# speedup vs baseline: 1.0045x; 1.0045x over previous
import functools

import jax
import jax.numpy as jnp
from jax import lax
from jax.experimental import pallas as pl
from jax.experimental.pallas import tpu as pltpu
from jax.experimental.pallas import tpu_sc as plsc

N = 10000
E = 320000
D_IN = 128
D_HID = 16
D_PAD = 16

NUM_CORES = 2
NUM_SUBCORES = 16
NW = NUM_CORES * NUM_SUBCORES

CHUNK = 256
NBUF = 4
K = NBUF * (-(-E // (NW * CHUNK * NBUF)))
G = K // NBUF
EW = NW * K * CHUNK
assert G % 2 == 0
N_PAD = ((N + NW * 8 - 1) // (NW * 8)) * (NW * 8) + NW * 8
RPT = N_PAD // NUM_SUBCORES

_mesh = plsc.VectorSubcoreMesh(core_axis_name="c", subcore_axis_name="s")


@functools.partial(
    pl.kernel,
    mesh=_mesh,
    out_type=[
        jax.ShapeDtypeStruct((NW * RPT,), jnp.float32),
        jax.ShapeDtypeStruct((NW * RPT, D_PAD), jnp.float32),
    ],
    scratch_types=[
        pltpu.VMEM((K, CHUNK), jnp.int32),
        pltpu.VMEM((2 * K, CHUNK), jnp.int32),
        pltpu.VMEM((CHUNK,), jnp.float32),
        pltpu.VMEM((RPT,), jnp.float32),
        pltpu.VMEM((2 * NBUF, CHUNK, D_PAD), jnp.float32),
        pltpu.VMEM((RPT, D_PAD), jnp.float32),
        pltpu.VMEM_SHARED((N_PAD,), jnp.float32),
        pltpu.VMEM_SHARED((N_PAD, D_PAD), jnp.float32),
        pltpu.VMEM_SHARED((N_PAD, D_PAD), jnp.float32),
        pltpu.SemaphoreType.DMA((2 * NBUF,)),
        pltpu.SemaphoreType.DMA,
        pltpu.SemaphoreType.DMA,
    ],
    compiler_params=pltpu.CompilerParams(use_tc_tiling_on_sc=False),
)
def _deg_agg_kernel(src_hbm, dst_hbm, h1_hbm, deg_hbm, out_hbm,
                    src_v, dst_v, ones_v, dval_v, rows_v, buf_v,
                    deg_sh, tab_sh, acc_sh, sem, dsem, tsem):
    c = lax.axis_index("c")
    s = lax.axis_index("s")
    w = c * NUM_SUBCORES + s
    soff = pl.multiple_of(s * RPT, 8)
    woff = pl.multiple_of(w * RPT, 8)

    pltpu.async_copy(h1_hbm.at[pl.ds(soff, RPT), :],
                     tab_sh.at[pl.ds(soff, RPT), :], tsem)

    pltpu.sync_copy(src_hbm.at[w], src_v)
    pltpu.sync_copy(dst_hbm.at[s], dst_v.at[pl.ds(0, K)])
    pltpu.sync_copy(dst_hbm.at[s + NUM_SUBCORES], dst_v.at[pl.ds(K, K)])
    for i in range(CHUNK // 16):
        ones_v[pl.ds(i * 16, 16)] = jnp.ones((16,), jnp.float32)

    def dzfill(i, _):
        dval_v[pl.ds(i * 16, 16)] = jnp.zeros((16,), jnp.float32)
        return _
    lax.fori_loop(0, RPT // 16, dzfill, None)
    pltpu.sync_copy(dval_v, deg_sh.at[pl.ds(soff, RPT)])

    def zfill(i, _):
        buf_v[i, :] = jnp.zeros((16,), jnp.float32)
        return _
    lax.fori_loop(0, RPT, zfill, None)
    pltpu.sync_copy(buf_v, acc_sh.at[pl.ds(soff, RPT), :])
    plsc.subcore_barrier()

    def dchunk(j, _):
        pltpu.async_copy(ones_v, deg_sh.at[dst_v.at[j]], dsem, add=True)

        @pl.when(j >= 8)
        def _():
            pltpu.make_async_copy(ones_v, deg_sh.at[dst_v.at[j - 8]], dsem).wait()
        return _
    lax.fori_loop(0, 2 * K, dchunk, None)
    for t in range(8):
        pltpu.make_async_copy(ones_v, deg_sh.at[dst_v.at[2 * K - 8 + t]],
                              dsem).wait()
    plsc.subcore_barrier()

    pltpu.sync_copy(deg_sh.at[pl.ds(soff, RPT)], dval_v)
    pltpu.sync_copy(dval_v, deg_hbm.at[pl.ds(woff, RPT)])

    def dis_fill(i, _):
        x = dval_v[pl.ds(i * 16, 16)] + 1.0
        ix = lax.bitcast_convert_type(x, jnp.int32)
        iy = jnp.int32(0x5F3759DF) - lax.shift_right_arithmetic(ix, 1)
        y = lax.bitcast_convert_type(iy, jnp.float32)
        for _t in range(4):
            y = y * (1.5 - 0.5 * x * y * y)
        dval_v[pl.ds(i * 16, 16)] = y
        return _
    lax.fori_loop(0, RPT // 16, dis_fill, None)

    pltpu.make_async_copy(h1_hbm.at[pl.ds(soff, RPT), :],
                          tab_sh.at[pl.ds(soff, RPT), :], tsem).wait()
    pltpu.sync_copy(tab_sh.at[pl.ds(soff, RPT), :], buf_v)

    def scale_block(i, _):
        dvec = dval_v[pl.ds(i * 16, 16)]
        for t in range(16):
            r = i * 16 + t
            buf_v[r, :] = buf_v[r, :] * dvec[t]
        return _
    lax.fori_loop(0, RPT // 16, scale_block, None)
    pltpu.sync_copy(buf_v, tab_sh.at[pl.ds(soff, RPT), :])
    plsc.subcore_barrier()

    def fire_g(j, b):
        pltpu.async_copy(tab_sh.at[src_v.at[j]], rows_v.at[b], sem.at[b])

    def wait_g(j, b):
        pltpu.make_async_copy(tab_sh.at[src_v.at[j]], rows_v.at[b],
                              sem.at[b]).wait()

    def fire_s(j, b):
        pltpu.async_copy(rows_v.at[b], acc_sh.at[dst_v.at[c * K + j]],
                         sem.at[b], add=True)

    def wait_s(j, b):
        pltpu.make_async_copy(rows_v.at[b], acc_sh.at[dst_v.at[c * K + j]],
                              sem.at[b]).wait()

    for b in range(NBUF):
        fire_g(b, b)

    def group2(i2, _):
        for p in (0, 1):
            gi = 2 * i2 + p
            pb = NBUF * p
            ob = NBUF * (1 - p)
            for b in range(NBUF):
                wait_g(gi * NBUF + b, pb + b)
            for b in range(NBUF):
                fire_s(gi * NBUF + b, pb + b)
            for b in range(NBUF):
                @pl.when(gi >= 1)
                def _(jp=(gi - 1) * NBUF + b, bb=ob + b):
                    wait_s(jp, bb)
            for b in range(NBUF):
                @pl.when(gi + 1 < G)
                def _(jn=(gi + 1) * NBUF + b, bb=ob + b):
                    fire_g(jn, bb)
        return _
    lax.fori_loop(0, G // 2, group2, None)
    for b in range(NBUF):
        wait_s((G - 1) * NBUF + b, NBUF * ((G - 1) % 2) + b)
    plsc.subcore_barrier()

    pltpu.sync_copy(acc_sh.at[pl.ds(soff, RPT), :], buf_v)
    pltpu.sync_copy(buf_v, out_hbm.at[pl.ds(woff, RPT), :])


@functools.partial(
    pl.kernel,
    mesh=_mesh,
    out_type=jax.ShapeDtypeStruct((NW * RPT, D_PAD), jnp.float32),
    scratch_types=[
        pltpu.VMEM((K, CHUNK), jnp.int32),
        pltpu.VMEM((K, CHUNK), jnp.int32),
        pltpu.VMEM((2 * NBUF, CHUNK, D_PAD), jnp.float32),
        pltpu.VMEM((RPT, D_PAD), jnp.float32),
        pltpu.VMEM_SHARED((N_PAD, D_PAD), jnp.float32),
        pltpu.VMEM_SHARED((N_PAD, D_PAD), jnp.float32),
        pltpu.SemaphoreType.DMA((2 * NBUF,)),
        pltpu.SemaphoreType.DMA,
    ],
    compiler_params=pltpu.CompilerParams(use_tc_tiling_on_sc=False),
)
def _agg_kernel(src_hbm, dst_hbm, table_hbm, out_hbm,
                src_v, dst_v, rows_v, buf_v, tab_sh, acc_sh, sem, tsem):
    c = lax.axis_index("c")
    s = lax.axis_index("s")
    w = c * NUM_SUBCORES + s
    soff = pl.multiple_of(s * RPT, 8)
    woff = pl.multiple_of(w * RPT, 8)

    pltpu.async_copy(table_hbm.at[pl.ds(soff, RPT), :],
                     tab_sh.at[pl.ds(soff, RPT), :], tsem)

    pltpu.sync_copy(src_hbm.at[w], src_v)
    pltpu.sync_copy(dst_hbm.at[w], dst_v)

    def zfill(i, _):
        buf_v[i, :] = jnp.zeros((16,), jnp.float32)
        return _
    lax.fori_loop(0, RPT, zfill, None)
    pltpu.sync_copy(buf_v, acc_sh.at[pl.ds(soff, RPT), :])
    pltpu.make_async_copy(table_hbm.at[pl.ds(soff, RPT), :],
                          tab_sh.at[pl.ds(soff, RPT), :], tsem).wait()
    plsc.subcore_barrier()

    def fire_g(j, b):
        pltpu.async_copy(tab_sh.at[src_v.at[j]], rows_v.at[b], sem.at[b])

    def wait_g(j, b):
        pltpu.make_async_copy(tab_sh.at[src_v.at[j]], rows_v.at[b],
                              sem.at[b]).wait()

    def fire_s(j, b):
        pltpu.async_copy(rows_v.at[b], acc_sh.at[dst_v.at[j]], sem.at[b],
                         add=True)

    def wait_s(j, b):
        pltpu.make_async_copy(rows_v.at[b], acc_sh.at[dst_v.at[j]],
                              sem.at[b]).wait()

    for b in range(NBUF):
        fire_g(b, b)

    def group2(i2, _):
        for p in (0, 1):
            gi = 2 * i2 + p
            pb = NBUF * p
            ob = NBUF * (1 - p)
            for b in range(NBUF):
                wait_g(gi * NBUF + b, pb + b)
            for b in range(NBUF):
                fire_s(gi * NBUF + b, pb + b)
            for b in range(NBUF):
                @pl.when(gi >= 1)
                def _(jp=(gi - 1) * NBUF + b, bb=ob + b):
                    wait_s(jp, bb)
            for b in range(NBUF):
                @pl.when(gi + 1 < G)
                def _(jn=(gi + 1) * NBUF + b, bb=ob + b):
                    fire_g(jn, bb)
        return _
    lax.fori_loop(0, G // 2, group2, None)
    for b in range(NBUF):
        wait_s((G - 1) * NBUF + b, NBUF * ((G - 1) % 2) + b)
    plsc.subcore_barrier()

    pltpu.sync_copy(acc_sh.at[pl.ds(soff, RPT), :], buf_v)
    pltpu.sync_copy(buf_v, out_hbm.at[pl.ds(woff, RPT), :])


def _tc1_body(x_ref, w_ref, h_ref):
    h_ref[...] = jnp.dot(x_ref[...], w_ref[...],
                         preferred_element_type=jnp.float32)


def _tc2_body(acc_ref, h1_ref, deg_ref, w2_ref, b1_ref, g2_ref):
    deg = deg_ref[0] + 1.0
    dis = lax.rsqrt(deg)
    g1 = h1_ref[...] * dis
    out1 = dis * (acc_ref[0] + acc_ref[1] + g1) + b1_ref[...]
    h = jnp.maximum(out1, 0.0)
    g2_ref[...] = jnp.dot(h, w2_ref[...], preferred_element_type=jnp.float32) * dis


def _tc3_body(acc_ref, g2_ref, deg_ref, b2_ref, o_ref):
    deg = deg_ref[0] + 1.0
    dis = lax.rsqrt(deg)
    o = dis * (acc_ref[0] + acc_ref[1] + g2_ref[...]) + b2_ref[...]
    col = lax.broadcasted_iota(jnp.int32, o.shape, 1)
    mask = col < 5
    om = jnp.where(mask, o, -1e30)
    m = jnp.max(om, axis=1, keepdims=True)
    e = jnp.where(mask, jnp.exp(o - m), 0.0)
    ssum = jnp.sum(e, axis=1, keepdims=True)
    o_ref[...] = o - m - jnp.log(ssum)


_f32 = jnp.float32


@jax.jit
def kernel(x, edge_index, W1, b1, W2, b2):
    pad_e = EW - E
    src = jnp.concatenate([edge_index[0], jnp.full((pad_e,), N, jnp.int32)])
    dst = jnp.concatenate([edge_index[1], jnp.full((pad_e,), N, jnp.int32)])
    src3 = src.reshape(NW, K, CHUNK)
    dst3 = dst.reshape(NW, K, CHUNK)

    xp = jnp.zeros((N_PAD, D_IN), _f32).at[:N].set(x)
    W2p = jnp.zeros((D_HID, D_PAD), _f32).at[:, :W2.shape[1]].set(W2)
    b2p = jnp.zeros((D_PAD,), _f32).at[:W2.shape[1]].set(b2)

    h1p = pl.pallas_call(
        _tc1_body,
        out_shape=jax.ShapeDtypeStruct((N_PAD, D_HID), _f32),
    )(xp, W1)

    degp, acc1f = _deg_agg_kernel(src3, dst3, h1p)
    degp3 = degp.reshape(NUM_CORES, N_PAD, 1)
    acc1p = acc1f.reshape(NUM_CORES, N_PAD, D_PAD)

    g2p = pl.pallas_call(
        _tc2_body,
        out_shape=jax.ShapeDtypeStruct((N_PAD, D_PAD), _f32),
    )(acc1p, h1p, degp3, W2p, b1)

    acc2p = _agg_kernel(src3, dst3, g2p).reshape(NUM_CORES, N_PAD, D_PAD)

    outp = pl.pallas_call(
        _tc3_body,
        out_shape=jax.ShapeDtypeStruct((N_PAD, D_PAD), _f32),
    )(acc2p, g2p, degp3, b2p)

    return outp[:N, :W2.shape[1]]


# speedup vs baseline: 1.3968x; 1.3905x over previous
import functools

import jax
import jax.numpy as jnp
from jax import lax
from jax.experimental import pallas as pl
from jax.experimental.pallas import tpu as pltpu
from jax.experimental.pallas import tpu_sc as plsc

N = 10000
E = 320000
D_IN = 128
D_HID = 16
D_PAD = 16

NUM_CORES = 2
NUM_SUBCORES = 16
NW = NUM_CORES * NUM_SUBCORES

CHUNK = 256
NBUF = 4
K = NBUF * (-(-E // (NW * CHUNK * NBUF)))
G = K // NBUF
EW = NW * K * CHUNK
assert G % 2 == 0
N_PAD = ((N + NW * 8 - 1) // (NW * 8)) * (NW * 8) + NW * 8
RPT = N_PAD // NUM_SUBCORES

_mesh = plsc.VectorSubcoreMesh(core_axis_name="c", subcore_axis_name="s")


@functools.partial(
    pl.kernel,
    mesh=_mesh,
    out_type=[
        jax.ShapeDtypeStruct((NW * RPT,), jnp.float32),
        jax.ShapeDtypeStruct((NW * RPT, D_PAD), jnp.float32),
    ],
    scratch_types=[
        pltpu.VMEM((K, CHUNK), jnp.int32),
        pltpu.VMEM((2 * K, CHUNK), jnp.int32),
        pltpu.VMEM((CHUNK,), jnp.float32),
        pltpu.VMEM((RPT,), jnp.float32),
        pltpu.VMEM((2 * NBUF, CHUNK, D_PAD), jnp.float32),
        pltpu.VMEM((RPT, D_PAD), jnp.float32),
        pltpu.VMEM_SHARED((N_PAD,), jnp.float32),
        pltpu.VMEM_SHARED((N_PAD, D_PAD), jnp.float32),
        pltpu.VMEM_SHARED((N_PAD, D_PAD), jnp.float32),
        pltpu.SemaphoreType.DMA((2 * NBUF,)),
        pltpu.SemaphoreType.DMA,
        pltpu.SemaphoreType.DMA,
    ],
    compiler_params=pltpu.CompilerParams(use_tc_tiling_on_sc=False),
)
def _deg_agg_kernel(src_hbm, dst_hbm, h1_hbm, deg_hbm, out_hbm,
                    src_v, dst_v, ones_v, dval_v, rows_v, buf_v,
                    deg_sh, tab_sh, acc_sh, sem, dsem, tsem):
    c = lax.axis_index("c")
    s = lax.axis_index("s")
    w = c * NUM_SUBCORES + s
    soff = pl.multiple_of(s * RPT, 8)
    woff = pl.multiple_of(w * RPT, 8)

    pltpu.async_copy(h1_hbm.at[pl.ds(soff, RPT), :],
                     tab_sh.at[pl.ds(soff, RPT), :], tsem)

    pltpu.sync_copy(src_hbm.at[w], src_v)
    pltpu.sync_copy(dst_hbm.at[s], dst_v.at[pl.ds(0, K)])
    pltpu.sync_copy(dst_hbm.at[s + NUM_SUBCORES], dst_v.at[pl.ds(K, K)])
    for i in range(CHUNK // 16):
        ones_v[pl.ds(i * 16, 16)] = jnp.ones((16,), jnp.float32)

    def dzfill(i, _):
        dval_v[pl.ds(i * 16, 16)] = jnp.zeros((16,), jnp.float32)
        return _
    lax.fori_loop(0, RPT // 16, dzfill, None)
    pltpu.sync_copy(dval_v, deg_sh.at[pl.ds(soff, RPT)])

    def zfill(i, _):
        buf_v[i, :] = jnp.zeros((16,), jnp.float32)
        return _
    lax.fori_loop(0, RPT, zfill, None)
    pltpu.sync_copy(buf_v, acc_sh.at[pl.ds(soff, RPT), :])
    plsc.subcore_barrier()

    def dchunk(j, _):
        pltpu.async_copy(ones_v, deg_sh.at[dst_v.at[j]], dsem, add=True)

        @pl.when(j >= 8)
        def _():
            pltpu.make_async_copy(ones_v, deg_sh.at[dst_v.at[j - 8]], dsem).wait()
        return _
    lax.fori_loop(0, 2 * K, dchunk, None)
    for t in range(8):
        pltpu.make_async_copy(ones_v, deg_sh.at[dst_v.at[2 * K - 8 + t]],
                              dsem).wait()
    plsc.subcore_barrier()

    pltpu.sync_copy(deg_sh.at[pl.ds(soff, RPT)], dval_v)
    pltpu.sync_copy(dval_v, deg_hbm.at[pl.ds(woff, RPT)])

    def dis_fill(i, _):
        x = dval_v[pl.ds(i * 16, 16)] + 1.0
        ix = lax.bitcast_convert_type(x, jnp.int32)
        iy = jnp.int32(0x5F3759DF) - lax.shift_right_arithmetic(ix, 1)
        y = lax.bitcast_convert_type(iy, jnp.float32)
        for _t in range(4):
            y = y * (1.5 - 0.5 * x * y * y)
        dval_v[pl.ds(i * 16, 16)] = y
        return _
    lax.fori_loop(0, RPT // 16, dis_fill, None)

    pltpu.make_async_copy(h1_hbm.at[pl.ds(soff, RPT), :],
                          tab_sh.at[pl.ds(soff, RPT), :], tsem).wait()
    pltpu.sync_copy(tab_sh.at[pl.ds(soff, RPT), :], buf_v)

    def scale_block(i, _):
        dvec = dval_v[pl.ds(i * 16, 16)]
        for t in range(16):
            r = i * 16 + t
            buf_v[r, :] = buf_v[r, :] * dvec[t]
        return _
    lax.fori_loop(0, RPT // 16, scale_block, None)
    pltpu.sync_copy(buf_v, tab_sh.at[pl.ds(soff, RPT), :])
    plsc.subcore_barrier()

    def fire_g(j, b):
        pltpu.async_copy(tab_sh.at[src_v.at[j]], rows_v.at[b], sem.at[b])

    def wait_g(j, b):
        pltpu.make_async_copy(tab_sh.at[src_v.at[j]], rows_v.at[b],
                              sem.at[b]).wait()

    def fire_s(j, b):
        pltpu.async_copy(rows_v.at[b], acc_sh.at[dst_v.at[c * K + j]],
                         sem.at[b], add=True)

    def wait_s(j, b):
        pltpu.make_async_copy(rows_v.at[b], acc_sh.at[dst_v.at[c * K + j]],
                              sem.at[b]).wait()

    for b in range(NBUF):
        fire_g(b, b)

    def group2(i2, _):
        for p in (0, 1):
            gi = 2 * i2 + p
            pb = NBUF * p
            ob = NBUF * (1 - p)
            for b in range(NBUF):
                wait_g(gi * NBUF + b, pb + b)
            for b in range(NBUF):
                fire_s(gi * NBUF + b, pb + b)
            for b in range(NBUF):
                @pl.when(gi >= 1)
                def _(jp=(gi - 1) * NBUF + b, bb=ob + b):
                    wait_s(jp, bb)
            for b in range(NBUF):
                @pl.when(gi + 1 < G)
                def _(jn=(gi + 1) * NBUF + b, bb=ob + b):
                    fire_g(jn, bb)
        return _
    lax.fori_loop(0, G // 2, group2, None)
    for b in range(NBUF):
        wait_s((G - 1) * NBUF + b, NBUF * ((G - 1) % 2) + b)
    plsc.subcore_barrier()

    pltpu.sync_copy(acc_sh.at[pl.ds(soff, RPT), :], buf_v)
    pltpu.sync_copy(buf_v, out_hbm.at[pl.ds(woff, RPT), :])


@functools.partial(
    pl.kernel,
    mesh=_mesh,
    out_type=jax.ShapeDtypeStruct((NW * RPT, D_PAD), jnp.float32),
    scratch_types=[
        pltpu.VMEM((K, CHUNK), jnp.int32),
        pltpu.VMEM((K, CHUNK), jnp.int32),
        pltpu.VMEM((2 * NBUF, CHUNK, D_PAD), jnp.float32),
        pltpu.VMEM((RPT, D_PAD), jnp.float32),
        pltpu.VMEM_SHARED((N_PAD, D_PAD), jnp.float32),
        pltpu.VMEM_SHARED((N_PAD, D_PAD), jnp.float32),
        pltpu.SemaphoreType.DMA((2 * NBUF,)),
        pltpu.SemaphoreType.DMA,
    ],
    compiler_params=pltpu.CompilerParams(use_tc_tiling_on_sc=False),
)
def _agg_kernel(src_hbm, dst_hbm, table_hbm, out_hbm,
                src_v, dst_v, rows_v, buf_v, tab_sh, acc_sh, sem, tsem):
    c = lax.axis_index("c")
    s = lax.axis_index("s")
    w = c * NUM_SUBCORES + s
    soff = pl.multiple_of(s * RPT, 8)
    woff = pl.multiple_of(w * RPT, 8)

    pltpu.async_copy(table_hbm.at[pl.ds(soff, RPT), :],
                     tab_sh.at[pl.ds(soff, RPT), :], tsem)

    pltpu.sync_copy(src_hbm.at[w], src_v)
    pltpu.sync_copy(dst_hbm.at[w], dst_v)

    def zfill(i, _):
        buf_v[i, :] = jnp.zeros((16,), jnp.float32)
        return _
    lax.fori_loop(0, RPT, zfill, None)
    pltpu.sync_copy(buf_v, acc_sh.at[pl.ds(soff, RPT), :])
    pltpu.make_async_copy(table_hbm.at[pl.ds(soff, RPT), :],
                          tab_sh.at[pl.ds(soff, RPT), :], tsem).wait()
    plsc.subcore_barrier()

    def fire_g(j, b):
        pltpu.async_copy(tab_sh.at[src_v.at[j]], rows_v.at[b], sem.at[b])

    def wait_g(j, b):
        pltpu.make_async_copy(tab_sh.at[src_v.at[j]], rows_v.at[b],
                              sem.at[b]).wait()

    def fire_s(j, b):
        pltpu.async_copy(rows_v.at[b], acc_sh.at[dst_v.at[j]], sem.at[b],
                         add=True)

    def wait_s(j, b):
        pltpu.make_async_copy(rows_v.at[b], acc_sh.at[dst_v.at[j]],
                              sem.at[b]).wait()

    for b in range(NBUF):
        fire_g(b, b)

    def group2(i2, _):
        for p in (0, 1):
            gi = 2 * i2 + p
            pb = NBUF * p
            ob = NBUF * (1 - p)
            for b in range(NBUF):
                wait_g(gi * NBUF + b, pb + b)
            for b in range(NBUF):
                fire_s(gi * NBUF + b, pb + b)
            for b in range(NBUF):
                @pl.when(gi >= 1)
                def _(jp=(gi - 1) * NBUF + b, bb=ob + b):
                    wait_s(jp, bb)
            for b in range(NBUF):
                @pl.when(gi + 1 < G)
                def _(jn=(gi + 1) * NBUF + b, bb=ob + b):
                    fire_g(jn, bb)
        return _
    lax.fori_loop(0, G // 2, group2, None)
    for b in range(NBUF):
        wait_s((G - 1) * NBUF + b, NBUF * ((G - 1) % 2) + b)
    plsc.subcore_barrier()

    pltpu.sync_copy(acc_sh.at[pl.ds(soff, RPT), :], buf_v)
    pltpu.sync_copy(buf_v, out_hbm.at[pl.ds(woff, RPT), :])


X = N_PAD // 8


def _tc1_body(x_ref, w_ref, h_ref):
    h = jnp.dot(x_ref[...], w_ref[...], preferred_element_type=jnp.float32)
    for g in range(8):
        h_ref[:, 16 * g:16 * (g + 1)] = h[g * X:(g + 1) * X, :]


def _tc2_body(acc_ref, h1_ref, deg_ref, bsel_ref, w2_ref, b1_ref, g2_ref):
    dis2d = lax.rsqrt(deg_ref[...] + 1.0)
    dis = jnp.dot(dis2d, bsel_ref[...],
                  preferred_element_type=jnp.float32)
    g1 = h1_ref[...] * dis
    out1 = dis * (acc_ref[0:X] + acc_ref[X:2 * X] + g1) + b1_ref[...]
    h = jnp.maximum(out1, 0.0)
    g2_ref[...] = jnp.dot(h, w2_ref[...], preferred_element_type=jnp.float32) * dis


def _tc3_body(acc_ref, g2_ref, deg_ref, bsel_ref, b2_ref, o_ref):
    dis2d = lax.rsqrt(deg_ref[...] + 1.0)
    dis = jnp.dot(dis2d, bsel_ref[...], preferred_element_type=jnp.float32)
    o_pk = dis * (acc_ref[0:X] + acc_ref[X:2 * X] + g2_ref[...]) + b2_ref[...]
    o = jnp.concatenate([o_pk[:, 16 * g:16 * (g + 1)] for g in range(8)], axis=0)
    col = lax.broadcasted_iota(jnp.int32, o.shape, 1)
    mask = col < 5
    om = jnp.where(mask, o, -1e30)
    m = jnp.max(om, axis=1, keepdims=True)
    e = jnp.where(mask, jnp.exp(o - m), 0.0)
    ssum = jnp.sum(e, axis=1, keepdims=True)
    o_ref[...] = o - m - jnp.log(ssum)


_f32 = jnp.float32


@jax.jit
def kernel(x, edge_index, W1, b1, W2, b2):
    pad_e = EW - E
    src = jnp.concatenate([edge_index[0], jnp.full((pad_e,), N, jnp.int32)])
    dst = jnp.concatenate([edge_index[1], jnp.full((pad_e,), N, jnp.int32)])
    src = 8 * (src % X) + src // X
    dst = 8 * (dst % X) + dst // X
    src3 = src.reshape(NW, K, CHUNK)
    dst3 = dst.reshape(NW, K, CHUNK)

    xp = jnp.zeros((N_PAD, D_IN), _f32).at[:N].set(x)
    W2p = jnp.zeros((D_HID, D_PAD), _f32).at[:, :W2.shape[1]].set(W2)
    b2p = jnp.zeros((D_PAD,), _f32).at[:W2.shape[1]].set(b2)
    W2blk = jnp.kron(jnp.eye(8, dtype=_f32), W2p)
    bsel = jnp.kron(jnp.eye(8, dtype=_f32), jnp.ones((1, D_PAD), _f32))
    b1t = jnp.tile(b1, 8)
    b2t = jnp.tile(b2p, 8)

    h1pk = pl.pallas_call(
        _tc1_body,
        out_shape=jax.ShapeDtypeStruct((X, 128), _f32),
    )(xp, W1)

    degp, acc1f = _deg_agg_kernel(src3, dst3, h1pk.reshape(N_PAD, D_PAD))
    deg2d = degp[:N_PAD].reshape(X, 8)
    acc1pk = acc1f.reshape(2 * X, 128)

    g2pk = pl.pallas_call(
        _tc2_body,
        out_shape=jax.ShapeDtypeStruct((X, 128), _f32),
    )(acc1pk, h1pk, deg2d, bsel, W2blk, b1t)

    acc2pk = _agg_kernel(src3, dst3, g2pk.reshape(N_PAD, D_PAD)).reshape(2 * X, 128)

    outp = pl.pallas_call(
        _tc3_body,
        out_shape=jax.ShapeDtypeStruct((N_PAD, D_PAD), _f32),
    )(acc2pk, g2pk, deg2d, bsel, b2t)

    return outp[:N, :W2.shape[1]]
